# baseline (speedup 1.0000x reference)
import functools

import jax
import jax.numpy as jnp
from jax import lax
from jax.experimental import pallas as pl
from jax.experimental.pallas import tpu as pltpu
from jax.experimental.pallas import tpu_sc as plsc

N = 10000
D = 256
E = 160000
NP = 10240
BLK = 1024
GRID = NP // BLK

NC = 2
NS = 16
COLS = 64
NPASS = 4
CHUNK = 128
KBUF = 8
NGRP = 10
NCHUNK = NGRP * KBUF
NCK = NCHUNK + KBUF
EPT_A = NCK * CHUNK
NCHUNK_D = 40
EPT_D = NCHUNK_D * CHUNK
E_PAD_D = NC * NS * EPT_D
DW = 16
ROWS_PT = NP // NS
RCHUNK = ROWS_PT // CHUNK

_MESH = plsc.VectorSubcoreMesh(
    core_axis_name="c", subcore_axis_name="s", num_cores=NC, num_subcores=NS)


@functools.partial(
    pl.kernel,
    out_type=[jax.ShapeDtypeStruct((NP, DW), jnp.float32)] * NC,
    mesh=_MESH,
    scratch_types=[
        pltpu.VMEM((NCHUNK_D, CHUNK), jnp.int32),
        pltpu.VMEM((CHUNK, DW), jnp.float32),
        pltpu.VMEM_SHARED((NP, DW), jnp.float32),
    ],
)
def _deg_kernel(dst_hbm, out0, out1, dst_v, buf, hist):
    c = lax.axis_index("c")
    s = lax.axis_index("s")
    wid = c * NS + s
    pltpu.sync_copy(dst_hbm.at[wid], dst_v)
    base = s * ROWS_PT

    def fill(val):
        def fbody(r, carry):
            buf[r] = jnp.full((DW,), val, jnp.float32)
            return carry
        lax.fori_loop(0, CHUNK, fbody, 0)

    def run(o_hbm):
        fill(0.0)

        def zbody(k, carry):
            pltpu.sync_copy(buf, hist.at[pl.ds(base + k * CHUNK, CHUNK)])
            return carry
        lax.fori_loop(0, RCHUNK, zbody, 0)
        plsc.subcore_barrier()

        fill(1.0)

        def ebody(j, carry):
            pltpu.sync_copy(buf, hist.at[dst_v.at[j]], add=True)
            return carry
        lax.fori_loop(0, NCHUNK_D, ebody, 0)
        plsc.subcore_barrier()

        def obody(k, carry):
            rows = pl.ds(base + k * CHUNK, CHUNK)
            pltpu.sync_copy(hist.at[rows], buf)
            pltpu.sync_copy(buf, o_hbm.at[rows])
            return carry
        lax.fori_loop(0, RCHUNK, obody, 0)

    @pl.when(c == 0)
    def _():
        run(out0)

    @pl.when(c == 1)
    def _():
        run(out1)


def _dense_body(seq1_ref, seq2_ref, deg0_ref, deg1_ref, w_ref,
                *out_refs):
    t_refs = out_refs[:8]
    dinv_ref = out_refs[8]
    deg = deg0_ref[...][:, 0] + deg1_ref[...][:, 0] + 1.0
    dinv = lax.rsqrt(deg)
    w = w_ref[...]
    g1 = jnp.dot(seq1_ref[...], w, preferred_element_type=jnp.float32)
    g1 = g1 * dinv[:, None]
    g2 = jnp.dot(seq2_ref[...], w, preferred_element_type=jnp.float32)
    g2 = g2 * dinv[:, None]
    for p in range(NPASS):
        t_refs[p][...] = g1[:, p * COLS:(p + 1) * COLS]
        t_refs[NPASS + p][...] = g2[:, p * COLS:(p + 1) * COLS]
    dinv_ref[...] = dinv


_tab = jax.ShapeDtypeStruct((NP, COLS), jnp.float32)
_dense_call = pl.pallas_call(
    _dense_body,
    grid=(GRID,),
    in_specs=[
        pl.BlockSpec((BLK, D), lambda i: (i, 0)),
        pl.BlockSpec((BLK, D), lambda i: (i, 0)),
        pl.BlockSpec((BLK, DW), lambda i: (i, 0)),
        pl.BlockSpec((BLK, DW), lambda i: (i, 0)),
        pl.BlockSpec((D, D), lambda i: (0, 0)),
    ],
    out_specs=[pl.BlockSpec((BLK, COLS), lambda i: (i, 0))] * 8
    + [pl.BlockSpec((BLK,), lambda i: (i,))],
    out_shape=[_tab] * 8 + [jax.ShapeDtypeStruct((NP,), jnp.float32)],
)


@functools.partial(
    pl.kernel,
    out_type=[_tab] * 8,
    mesh=_MESH,
    scratch_types=[
        pltpu.VMEM((NCK, CHUNK), jnp.int32),
        pltpu.VMEM((NCK, CHUNK), jnp.int32),
    ]
    + [pltpu.VMEM((CHUNK, COLS), jnp.float32)] * KBUF
    + [pltpu.VMEM_SHARED((NP, COLS), jnp.float32)]
    + [pltpu.SemaphoreType.DMA] * (2 * KBUF),
    compiler_params=pltpu.CompilerParams(use_tc_tiling_on_sc=False),
)
def _agg_kernel(t0, t1, t2, t3, t4, t5, t6, t7, src_hbm, dst_hbm,
                o0, o1, o2, o3, o4, o5, o6, o7,
                src_v, dst_v, *rest):
    bufs = rest[:KBUF]
    acc = rest[KBUF]
    gsems = rest[KBUF + 1:KBUF + 1 + KBUF]
    ssems = rest[KBUF + 1 + KBUF:]
    c = lax.axis_index("c")
    s = lax.axis_index("s")
    pltpu.sync_copy(src_hbm.at[s], src_v)
    pltpu.sync_copy(dst_hbm.at[s], dst_v)
    base = s * ROWS_PT
    buf0 = bufs[0]

    def run_pass(t_hbm, o_hbm):
        def ibody(k, carry):
            rows = pl.ds(base + k * CHUNK, CHUNK)
            pltpu.sync_copy(t_hbm.at[rows], buf0)
            pltpu.sync_copy(buf0, acc.at[rows])
            return carry
        lax.fori_loop(0, RCHUNK, ibody, 0)
        plsc.subcore_barrier()

        for b in range(KBUF):
            pltpu.async_copy(t_hbm.at[src_v.at[b]], bufs[b], gsems[b])

        def gbody(g, carry):
            jb = g * KBUF
            descs = []
            for b in range(KBUF):
                pltpu.make_async_copy(t_hbm.at[src_v.at[jb + b]],
                                      bufs[b], gsems[b]).wait()
                descs.append(
                    pltpu.async_copy(bufs[b], acc.at[dst_v.at[jb + b]],
                                     ssems[b], add=True))
            for b in range(KBUF):
                descs[b].wait()
                pltpu.async_copy(t_hbm.at[src_v.at[jb + KBUF + b]],
                                 bufs[b], gsems[b])
            return carry
        lax.fori_loop(0, NGRP, gbody, 0)
        for b in range(KBUF):
            pltpu.make_async_copy(t_hbm.at[src_v.at[NCHUNK + b]],
                                  bufs[b], gsems[b]).wait()
        plsc.subcore_barrier()

        def obody(k, carry):
            rows = pl.ds(base + k * CHUNK, CHUNK)
            pltpu.sync_copy(acc.at[rows], buf0)
            pltpu.sync_copy(buf0, o_hbm.at[rows])
            return carry
        lax.fori_loop(0, RCHUNK, obody, 0)

    @pl.when(c == 0)
    def _():
        run_pass(t0, o0)
        run_pass(t1, o1)
        run_pass(t2, o2)
        run_pass(t3, o3)

    @pl.when(c == 1)
    def _():
        run_pass(t4, o4)
        run_pass(t5, o5)
        run_pass(t6, o6)
        run_pass(t7, o7)


def _final_body(*refs):
    o_refs = refs[:8]
    (dinv_ref, h3_ref, mask_ref, bg_ref, wl_ref, bl_ref,
     sc1_ref, sc2_ref) = refs[8:]
    dinv = dinv_ref[...]
    wsum = jnp.sum(wl_ref[...], axis=1)
    bsum = jnp.sum(bl_ref[...])
    bg = bg_ref[...]
    acc1 = jnp.concatenate([o_refs[p][...] for p in range(NPASS)], axis=1)
    h1 = jnp.maximum(acc1 * dinv[:, None] + bg[None, :], 0.0)
    acc2 = jnp.concatenate([o_refs[NPASS + p][...] for p in range(NPASS)],
                           axis=1)
    h2 = jnp.maximum(acc2 * dinv[:, None] + bg[None, :], 0.0)
    h2 = jnp.where(mask_ref[...][:, None] > 0.0, h3_ref[...], h2)
    sc1_ref[...] = jnp.sum(h1 * wsum[None, :], axis=1) + bsum
    sc2_ref[...] = jnp.sum(h2 * wsum[None, :], axis=1) + bsum


_final_call = pl.pallas_call(
    _final_body,
    grid=(GRID,),
    in_specs=[pl.BlockSpec((BLK, COLS), lambda i: (i, 0))] * 8
    + [
        pl.BlockSpec((BLK,), lambda i: (i,)),
        pl.BlockSpec((BLK, D), lambda i: (i, 0)),
        pl.BlockSpec((BLK,), lambda i: (i,)),
        pl.BlockSpec((D,), lambda i: (0,)),
        pl.BlockSpec((D, D), lambda i: (0, 0)),
        pl.BlockSpec((D,), lambda i: (0,)),
    ],
    out_specs=[
        pl.BlockSpec((BLK,), lambda i: (i,)),
        pl.BlockSpec((BLK,), lambda i: (i,)),
    ],
    out_shape=[jax.ShapeDtypeStruct((NP,), jnp.float32),
               jax.ShapeDtypeStruct((NP,), jnp.float32)],
)


def kernel(seq1, seq2, h_3, edge_index, W_gcn, b_gcn, W_lin, b_lin):
    src = edge_index[0]
    dst = edge_index[1]
    fill_d = jnp.full((E_PAD_D - E,), N, dtype=jnp.int32)
    dst_d = jnp.concatenate([dst, fill_d]).reshape(NC * NS, NCHUNK_D, CHUNK)

    deg0, deg1 = _deg_kernel(dst_d)

    seq1p = jnp.pad(seq1, ((0, NP - N), (0, 0)))
    seq2p = jnp.pad(seq2, ((0, NP - N), (0, 0)))
    dense_out = _dense_call(seq1p, seq2p, deg0, deg1, W_gcn)
    tabs, dinvp = dense_out[:8], dense_out[8]

    fill_tile = jnp.full((NS, KBUF * CHUNK), N, dtype=jnp.int32)
    fill_a = jnp.full((NS * NCHUNK * CHUNK - E,), N, dtype=jnp.int32)

    def _layout(ix):
        body = jnp.concatenate([ix, fill_a]).reshape(NS, NCHUNK * CHUNK)
        return jnp.concatenate([body, fill_tile], axis=1).reshape(
            NS, NCK, CHUNK)

    src_a = _layout(src)
    dst_a = _layout(dst)
    outs = _agg_kernel(*tabs, src_a, dst_a)

    s = jax.random.uniform(jax.random.key(42), (N,), dtype=jnp.float32)
    maskp = jnp.pad((s > 0.5).astype(jnp.float32), (0, NP - N))
    h3p = jnp.pad(h_3, ((0, NP - N), (0, 0)))
    sc1p, sc2p = _final_call(*outs, dinvp, h3p, maskp,
                             b_gcn, W_lin, b_lin)
    return jnp.concatenate([sc1p[:N], sc2p[:N]])


# speedup vs baseline: 2.6387x; 2.6387x over previous
import functools

import jax
import jax.numpy as jnp
from jax import lax
from jax.experimental import pallas as pl
from jax.experimental.pallas import tpu as pltpu
from jax.experimental.pallas import tpu_sc as plsc

N = 10000
D = 256
E = 160000
NP = 10240
BLK = 1024
GRID = NP // BLK

NC = 2
NS = 16
COLS = 128
CHUNK = 128
KBUF = 2
NGRP = 40
NGRP_P = NGRP + 1
NCHUNK = NGRP * KBUF
EPT_A = NGRP_P * KBUF * CHUNK
E_PAD_A = NS * EPT_A
NCHUNK_D = 40
EPT_D = NCHUNK_D * CHUNK
E_PAD_D = NC * NS * EPT_D
DW = 16
ROWS_PT = NP // NS
RCHUNK = ROWS_PT // CHUNK

_MESH = plsc.VectorSubcoreMesh(
    core_axis_name="c", subcore_axis_name="s", num_cores=NC, num_subcores=NS)


@functools.partial(
    pl.kernel,
    out_type=[jax.ShapeDtypeStruct((NP, DW), jnp.float32)] * NC,
    mesh=_MESH,
    scratch_types=[
        pltpu.VMEM((NCHUNK_D, CHUNK), jnp.int32),
        pltpu.VMEM((CHUNK, DW), jnp.float32),
        pltpu.VMEM_SHARED((NP, DW), jnp.float32),
    ],
)
def _deg_kernel(dst_hbm, out0, out1, dst_v, buf, hist):
    c = lax.axis_index("c")
    s = lax.axis_index("s")
    wid = c * NS + s
    pltpu.sync_copy(dst_hbm.at[wid], dst_v)
    base = s * ROWS_PT

    def fill(val):
        def fbody(r, carry):
            buf[r] = jnp.full((DW,), val, jnp.float32)
            return carry
        lax.fori_loop(0, CHUNK, fbody, 0)

    def run(o_hbm):
        fill(0.0)

        def zbody(k, carry):
            pltpu.sync_copy(buf, hist.at[pl.ds(base + k * CHUNK, CHUNK)])
            return carry
        lax.fori_loop(0, RCHUNK, zbody, 0)
        plsc.subcore_barrier()

        fill(1.0)

        def ebody(j, carry):
            pltpu.sync_copy(buf, hist.at[dst_v.at[j]], add=True)
            return carry
        lax.fori_loop(0, NCHUNK_D, ebody, 0)
        plsc.subcore_barrier()

        def obody(k, carry):
            rows = pl.ds(base + k * CHUNK, CHUNK)
            pltpu.sync_copy(hist.at[rows], buf)
            pltpu.sync_copy(buf, o_hbm.at[rows])
            return carry
        lax.fori_loop(0, RCHUNK, obody, 0)

    @pl.when(c == 0)
    def _():
        run(out0)

    @pl.when(c == 1)
    def _():
        run(out1)


def _dense_body(seq1_ref, seq2_ref, deg0_ref, deg1_ref, w_ref,
                t0_ref, t1_ref, t2_ref, t3_ref, dinv_ref):
    deg = deg0_ref[...][:, 0] + deg1_ref[...][:, 0] + 1.0
    dinv = lax.rsqrt(deg)
    w = w_ref[...]
    g1 = jnp.dot(seq1_ref[...], w, preferred_element_type=jnp.float32)
    g1 = g1 * dinv[:, None]
    g2 = jnp.dot(seq2_ref[...], w, preferred_element_type=jnp.float32)
    g2 = g2 * dinv[:, None]
    t0_ref[...] = g1[:, :COLS]
    t1_ref[...] = g1[:, COLS:]
    t2_ref[...] = g2[:, :COLS]
    t3_ref[...] = g2[:, COLS:]
    dinv_ref[...] = dinv


_tab = jax.ShapeDtypeStruct((NP, COLS), jnp.float32)
_dense_call = pl.pallas_call(
    _dense_body,
    grid=(GRID,),
    in_specs=[
        pl.BlockSpec((BLK, D), lambda i: (i, 0)),
        pl.BlockSpec((BLK, D), lambda i: (i, 0)),
        pl.BlockSpec((BLK, DW), lambda i: (i, 0)),
        pl.BlockSpec((BLK, DW), lambda i: (i, 0)),
        pl.BlockSpec((D, D), lambda i: (0, 0)),
    ],
    out_specs=[pl.BlockSpec((BLK, COLS), lambda i: (i, 0))] * 4
    + [pl.BlockSpec((BLK,), lambda i: (i,))],
    out_shape=[_tab] * 4 + [jax.ShapeDtypeStruct((NP,), jnp.float32)],
)


@functools.partial(
    pl.kernel,
    out_type=[_tab] * 4,
    mesh=_MESH,
    scratch_types=[
        pltpu.VMEM((2 * KBUF, CHUNK), jnp.int32),
        pltpu.VMEM((2 * KBUF, CHUNK), jnp.int32),
    ]
    + [pltpu.VMEM((CHUNK, COLS), jnp.float32)] * KBUF
    + [pltpu.VMEM_SHARED((NP, COLS), jnp.float32)]
    + [pltpu.SemaphoreType.DMA] * (2 * KBUF + 2),
)
def _agg_kernel(t0, t1, t2, t3, src_hbm, dst_hbm,
                o0, o1, o2, o3, sring, dring, *rest):
    bufs = rest[:KBUF]
    acc = rest[KBUF]
    gsems = rest[KBUF + 1:KBUF + 1 + KBUF]
    ssems = rest[KBUF + 1 + KBUF:KBUF + 1 + 2 * KBUF]
    isem_s = rest[KBUF + 1 + 2 * KBUF]
    isem_d = rest[KBUF + 2 + 2 * KBUF]
    c = lax.axis_index("c")
    s = lax.axis_index("s")
    base = s * ROWS_PT

    def rows_of(k):
        return pl.ds(base + k * CHUNK, CHUNK)

    def run_pass(t_hbm, o_hbm):
        rd = [pltpu.async_copy(t_hbm.at[rows_of(0)], bufs[0], gsems[0]),
              None]
        for k in range(RCHUNK):
            b = k & 1
            if k + 1 < RCHUNK:
                rd[1 - b] = pltpu.async_copy(t_hbm.at[rows_of(k + 1)],
                                             bufs[1 - b], gsems[1 - b])
            rd[b].wait()
            pltpu.sync_copy(bufs[b], acc.at[rows_of(k)])
        plsc.subcore_barrier()

        pltpu.async_copy(src_hbm.at[s, 0], sring.at[pl.ds(0, KBUF)], isem_s)
        pltpu.async_copy(dst_hbm.at[s, 0], dring.at[pl.ds(0, KBUF)], isem_d)

        def gbody(g, carry):
            p = lax.rem(g, 2) * KBUF
            pn = KBUF - p
            pltpu.make_async_copy(src_hbm.at[s, g],
                                  sring.at[pl.ds(p, KBUF)], isem_s).wait()
            pltpu.make_async_copy(dst_hbm.at[s, g],
                                  dring.at[pl.ds(p, KBUF)], isem_d).wait()
            pltpu.async_copy(src_hbm.at[s, g + 1],
                             sring.at[pl.ds(pn, KBUF)], isem_s)
            pltpu.async_copy(dst_hbm.at[s, g + 1],
                             dring.at[pl.ds(pn, KBUF)], isem_d)
            gds = []
            for b in range(KBUF):
                gds.append(
                    pltpu.async_copy(t_hbm.at[sring.at[p + b]],
                                     bufs[b], gsems[b]))
            sds = []
            for b in range(KBUF):
                gds[b].wait()
                sds.append(
                    pltpu.async_copy(bufs[b], acc.at[dring.at[p + b]],
                                     ssems[b], add=True))
            for b in range(KBUF):
                sds[b].wait()
            return carry
        lax.fori_loop(0, NGRP, gbody, 0)
        pf = lax.rem(jnp.int32(NGRP), 2) * KBUF
        pltpu.make_async_copy(src_hbm.at[s, NGRP],
                              sring.at[pl.ds(pf, KBUF)], isem_s).wait()
        pltpu.make_async_copy(dst_hbm.at[s, NGRP],
                              dring.at[pl.ds(pf, KBUF)], isem_d).wait()
        plsc.subcore_barrier()

        rd = [pltpu.async_copy(acc.at[rows_of(0)], bufs[0], gsems[0]),
              None]
        wd = [None, None]
        for k in range(RCHUNK):
            b = k & 1
            if k + 1 < RCHUNK:
                if wd[1 - b] is not None:
                    wd[1 - b].wait()
                rd[1 - b] = pltpu.async_copy(acc.at[rows_of(k + 1)],
                                             bufs[1 - b], gsems[1 - b])
            rd[b].wait()
            wd[b] = pltpu.async_copy(bufs[b], o_hbm.at[rows_of(k)],
                                     ssems[b])
        for b in range(2):
            if wd[b] is not None:
                wd[b].wait()

    @pl.when(c == 0)
    def _():
        run_pass(t0, o0)
        run_pass(t1, o1)

    @pl.when(c == 1)
    def _():
        run_pass(t2, o2)
        run_pass(t3, o3)


def _final_body(o0_ref, o1_ref, o2_ref, o3_ref, dinv_ref, h3_ref, mask_ref,
                bg_ref, wl_ref, bl_ref, sc1_ref, sc2_ref):
    dinv = dinv_ref[...]
    wsum = jnp.sum(wl_ref[...], axis=1)
    bsum = jnp.sum(bl_ref[...])
    bg = bg_ref[...]
    acc1 = jnp.concatenate([o0_ref[...], o1_ref[...]], axis=1)
    h1 = jnp.maximum(acc1 * dinv[:, None] + bg[None, :], 0.0)
    acc2 = jnp.concatenate([o2_ref[...], o3_ref[...]], axis=1)
    h2 = jnp.maximum(acc2 * dinv[:, None] + bg[None, :], 0.0)
    h2 = jnp.where(mask_ref[...][:, None] > 0.0, h3_ref[...], h2)
    sc1_ref[...] = jnp.sum(h1 * wsum[None, :], axis=1) + bsum
    sc2_ref[...] = jnp.sum(h2 * wsum[None, :], axis=1) + bsum


_final_call = pl.pallas_call(
    _final_body,
    grid=(GRID,),
    in_specs=[pl.BlockSpec((BLK, COLS), lambda i: (i, 0))] * 4
    + [
        pl.BlockSpec((BLK,), lambda i: (i,)),
        pl.BlockSpec((BLK, D), lambda i: (i, 0)),
        pl.BlockSpec((BLK,), lambda i: (i,)),
        pl.BlockSpec((D,), lambda i: (0,)),
        pl.BlockSpec((D, D), lambda i: (0, 0)),
        pl.BlockSpec((D,), lambda i: (0,)),
    ],
    out_specs=[
        pl.BlockSpec((BLK,), lambda i: (i,)),
        pl.BlockSpec((BLK,), lambda i: (i,)),
    ],
    out_shape=[jax.ShapeDtypeStruct((NP,), jnp.float32),
               jax.ShapeDtypeStruct((NP,), jnp.float32)],
)


def kernel(seq1, seq2, h_3, edge_index, W_gcn, b_gcn, W_lin, b_lin):
    src = edge_index[0]
    dst = edge_index[1]
    fill_d = jnp.full((E_PAD_D - E,), N, dtype=jnp.int32)
    dst_d = jnp.concatenate([dst, fill_d]).reshape(NC * NS, NCHUNK_D, CHUNK)

    deg0, deg1 = _deg_kernel(dst_d)

    seq1p = jnp.pad(seq1, ((0, NP - N), (0, 0)))
    seq2p = jnp.pad(seq2, ((0, NP - N), (0, 0)))
    t0, t1, t2, t3, dinvp = _dense_call(seq1p, seq2p, deg0, deg1, W_gcn)

    fill_tile = jnp.full((NS, KBUF * CHUNK), N, dtype=jnp.int32)
    fill_a = jnp.full((NS * NCHUNK * CHUNK - E,), N, dtype=jnp.int32)

    def _layout(ix):
        body = jnp.concatenate([ix, fill_a]).reshape(NS, NCHUNK * CHUNK)
        return jnp.concatenate([body, fill_tile], axis=1).reshape(
            NS, NGRP_P, KBUF, CHUNK)

    src_a = _layout(src)
    dst_a = _layout(dst)
    o0, o1, o2, o3 = _agg_kernel(t0, t1, t2, t3, src_a, dst_a)

    s = jax.random.uniform(jax.random.key(42), (N,), dtype=jnp.float32)
    maskp = jnp.pad((s > 0.5).astype(jnp.float32), (0, NP - N))
    h3p = jnp.pad(h_3, ((0, NP - N), (0, 0)))
    sc1p, sc2p = _final_call(o0, o1, o2, o3, dinvp, h3p, maskp,
                             b_gcn, W_lin, b_lin)
    return jnp.concatenate([sc1p[:N], sc2p[:N]])
